# baseline (speedup 1.0000x reference)
import functools

import jax
import jax.numpy as jnp
from jax import lax
from jax.experimental import pallas as pl
from jax.experimental.pallas import tpu as pltpu
from jax.experimental.pallas import tpu_sc as plsc

VOCAB = 100000
EMBED = 64
BATCH = 4096

BM = 512
BN = 4096


@functools.cache
def _sc_gather():
    info = plsc.get_sparse_core_info()
    nc, ns = info.num_cores, info.num_subcores
    nw = nc * ns
    b_per_w = BATCH // nw
    mesh = plsc.VectorSubcoreMesh(core_axis_name="c", subcore_axis_name="s")

    @functools.partial(
        pl.kernel,
        mesh=mesh,
        out_type=jax.ShapeDtypeStruct((BATCH, EMBED), jnp.float32),
        scratch_types=[
            pltpu.VMEM((b_per_w,), jnp.int32),
            pltpu.VMEM((b_per_w, EMBED), jnp.float32),
            pltpu.SemaphoreType.DMA,
        ],
        compiler_params=pltpu.CompilerParams(use_tc_tiling_on_sc=False),
    )
    def gather(table_hbm, idx_hbm, out_hbm, idx_v, rows_v, sem):
        wid = lax.axis_index("s") * nc + lax.axis_index("c")
        base = wid * b_per_w
        pltpu.sync_copy(idx_hbm.at[pl.ds(base, b_per_w)], idx_v)
        pltpu.async_copy(table_hbm.at[idx_v], rows_v, sem).wait()
        pltpu.sync_copy(rows_v, out_hbm.at[pl.ds(base, b_per_w)])

    return gather


def _mm_body(emb_ref, wt_ref, b_ref, out_ref):
    out_ref[...] = lax.dot_general(
        emb_ref[...], wt_ref[...],
        (((1,), (0,)), ((), ())),
        preferred_element_type=jnp.float32,
    ) + b_ref[...]


def _tc_matmul(embedded, WT, b):
    grid = (BATCH // BM, pl.cdiv(VOCAB, BN))
    return pl.pallas_call(
        _mm_body,
        grid=grid,
        in_specs=[
            pl.BlockSpec((BM, EMBED), lambda i, j: (i, 0)),
            pl.BlockSpec((EMBED, BN), lambda i, j: (0, j)),
            pl.BlockSpec((1, BN), lambda i, j: (0, j)),
        ],
        out_specs=pl.BlockSpec((BM, BN), lambda i, j: (i, j)),
        out_shape=jax.ShapeDtypeStruct((BATCH, VOCAB), jnp.float32),
        compiler_params=pltpu.CompilerParams(
            dimension_semantics=("parallel", "parallel"),
        ),
    )(embedded, WT, b.reshape(1, VOCAB))


def kernel(inputs, emb_table, W, b):
    embedded = _sc_gather()(emb_table, inputs)
    WT = W.T.astype(jnp.bfloat16)
    return _tc_matmul(embedded.astype(jnp.bfloat16), WT, b)


# speedup vs baseline: 1.0036x; 1.0036x over previous
import functools

import jax
import jax.numpy as jnp
from jax import lax
from jax.experimental import pallas as pl
from jax.experimental.pallas import tpu as pltpu
from jax.experimental.pallas import tpu_sc as plsc

VOCAB = 100000
EMBED = 64
BATCH = 4096

BM = 512
BN = 4096
NBUF = 4
JB = BATCH // BM
JN = pl.cdiv(VOCAB, BN)
TAIL = VOCAB - (JN - 1) * BN
VPAD = JN * BN
NSTEPS = JB * JN


@functools.cache
def _sc_gather():
    info = plsc.get_sparse_core_info()
    nc, ns = info.num_cores, info.num_subcores
    nw = nc * ns
    b_per_w = BATCH // nw
    mesh = plsc.VectorSubcoreMesh(core_axis_name="c", subcore_axis_name="s")

    @functools.partial(
        pl.kernel,
        mesh=mesh,
        out_type=jax.ShapeDtypeStruct((BATCH, EMBED), jnp.float32),
        scratch_types=[
            pltpu.VMEM((b_per_w,), jnp.int32),
            pltpu.VMEM((b_per_w, EMBED), jnp.float32),
            pltpu.SemaphoreType.DMA,
        ],
        compiler_params=pltpu.CompilerParams(use_tc_tiling_on_sc=False),
    )
    def gather(table_hbm, idx_hbm, out_hbm, idx_v, rows_v, sem):
        wid = lax.axis_index("s") * nc + lax.axis_index("c")
        base = wid * b_per_w
        pltpu.sync_copy(idx_hbm.at[pl.ds(base, b_per_w)], idx_v)
        pltpu.async_copy(table_hbm.at[idx_v], rows_v, sem).wait()
        pltpu.sync_copy(rows_v, out_hbm.at[pl.ds(base, b_per_w)])

    return gather


def _full_copy(acc, out_hbm, sems, s, i, j):
    return pltpu.make_async_copy(
        acc.at[s],
        out_hbm.at[pl.ds(i * BM, BM), pl.ds(j * BN, BN)],
        sems.at[s],
    )


def _tail_copy(acc_t, out_hbm, sem_t, i):
    return pltpu.make_async_copy(
        acc_t,
        out_hbm.at[pl.ds(i * BM, BM), pl.ds((JN - 1) * BN, TAIL)],
        sem_t,
    )


def _mm_body(emb_ref, wt_ref, b_ref, out_hbm, acc, acc_t, sems, sem_t):
    i = pl.program_id(0)
    j = pl.program_id(1)
    t = i * JN + j

    res = lax.dot_general(
        emb_ref[...], wt_ref[...],
        (((1,), (0,)), ((), ())),
        preferred_element_type=jnp.float32,
    ) + b_ref[...]

    for s in range(NBUF):
        @pl.when(jnp.logical_and(j < JN - 1, lax.rem(j, NBUF) == s))
        def _(s=s):
            @pl.when(jnp.logical_or(i > 0, j >= NBUF))
            def _():
                _full_copy(acc, out_hbm, sems, s, 0, 0).wait()

            acc[s] = res
            _full_copy(acc, out_hbm, sems, s, i, j).start(priority=s % 2)

    @pl.when(j == JN - 1)
    def _():
        @pl.when(i > 0)
        def _():
            _tail_copy(acc_t, out_hbm, sem_t, 0).wait()

        acc_t[...] = res[:, :TAIL]
        _tail_copy(acc_t, out_hbm, sem_t, i).start()

    @pl.when(t == NSTEPS - 1)
    def _():
        for s in range(NBUF):
            _full_copy(acc, out_hbm, sems, s, 0, 0).wait()
        _tail_copy(acc_t, out_hbm, sem_t, 0).wait()


def _tc_matmul(embedded, WT, b):
    return pl.pallas_call(
        _mm_body,
        grid=(JB, JN),
        in_specs=[
            pl.BlockSpec((BM, EMBED), lambda i, j: (i, 0)),
            pl.BlockSpec((EMBED, BN), lambda i, j: (0, j)),
            pl.BlockSpec((1, BN), lambda i, j: (0, j)),
        ],
        out_specs=pl.BlockSpec(memory_space=pl.ANY),
        out_shape=jax.ShapeDtypeStruct((BATCH, VOCAB), jnp.float32),
        scratch_shapes=[
            pltpu.VMEM((NBUF, BM, BN), jnp.float32),
            pltpu.VMEM((BM, TAIL), jnp.float32),
            pltpu.SemaphoreType.DMA((NBUF,)),
            pltpu.SemaphoreType.DMA,
        ],
        compiler_params=pltpu.CompilerParams(
            dimension_semantics=("arbitrary", "arbitrary"),
            vmem_limit_bytes=100 * 1024 * 1024,
        ),
    )(embedded, WT, b.reshape(1, VPAD))


def kernel(inputs, emb_table, W, b):
    embedded = _sc_gather()(emb_table, inputs)
    WT = jnp.pad(W.T.astype(jnp.bfloat16), ((0, 0), (0, VPAD - VOCAB)))
    bp = jnp.pad(b, (0, VPAD - VOCAB))
    return _tc_matmul(embedded.astype(jnp.bfloat16), WT, bp)
